# speedup vs baseline: 6.4645x; 6.4645x over previous
import jax
import jax.numpy as jnp
from jax.experimental import pallas as pl
from jax.experimental.pallas import tpu as pltpu

B = 16
T = 16384
N = 32
LMAX = 2048
BN = B * N
CHUNK = 512
NCHUNK = T // CHUNK
RESCALE = 8


def _crf_body(cu_ref, em_ref, tags_ref, src_ref, t2_ref, start_ref, end_ref,
              wbd_ref, ones_ref, sel_ref, len_ref, sexp_ref, eexp_ref,
              out_ref, escr):
    f32 = jnp.float32
    bf16 = jnp.bfloat16

    for b in range(B):
        cu_b = cu_ref[b]
        escr[:, b * N:(b + 1) * N] = jnp.exp(em_ref[pl.ds(cu_b, LMAX), :])

    def score_chunk(k, acc):
        t0 = k * CHUNK
        em_c = em_ref[pl.ds(t0, CHUNK), :]
        tg = tags_ref[pl.ds(t0, CHUNK), :]
        sr = src_ref[pl.ds(t0, CHUNK), :]
        lane = jax.lax.broadcasted_iota(jnp.int32, (CHUNK, N), 1)
        oh_tag = tg == lane
        oh_src = (sr == lane).astype(f32)
        em_sc = jnp.sum(jnp.where(oh_tag, em_c, 0.0), axis=1, keepdims=True)
        trans_row = jax.lax.dot_general(
            oh_src, t2_ref[:, :], (((1,), (0,)), ((), ())),
            preferred_element_type=f32)
        trans_sc = jnp.sum(jnp.where(oh_tag, trans_row, 0.0), axis=1,
                           keepdims=True)
        start_sc = jnp.sum(jnp.where(oh_tag, start_ref[:, :], 0.0), axis=1,
                           keepdims=True)
        end_sc = jnp.sum(jnp.where(oh_tag, end_ref[:, :], 0.0), axis=1,
                         keepdims=True)
        tvec = t0 + jax.lax.broadcasted_iota(jnp.int32, (CHUNK, 1), 0)
        isf = tvec == cu_ref[0]
        isl = tvec == (cu_ref[1] - 1)
        bid = jnp.zeros((CHUNK, 1), jnp.int32)
        for b in range(1, B):
            isf = isf | (tvec == cu_ref[b])
            isl = isl | (tvec == (cu_ref[b + 1] - 1))
            bid = bid + (tvec >= cu_ref[b]).astype(jnp.int32)
        tok = (em_sc + jnp.where(isf, start_sc, trans_sc)
               + jnp.where(isl, end_sc, 0.0))
        oh_b = bid == jax.lax.broadcasted_iota(jnp.int32, (CHUNK, B), 1)
        part = jnp.sum(jnp.where(oh_b, tok, 0.0), axis=0, keepdims=True)
        return acc + part

    scores = jax.lax.fori_loop(0, NCHUNK, score_chunk,
                               jnp.zeros((1, B), f32))

    dot_dims = (((1,), (0,)), ((), ()))

    def step_block(o, carry):
        alpha, acc = carry
        for u in range(RESCALE):
            i = o * RESCALE + u + 1
            e_i = escr[pl.ds(jnp.minimum(i, LMAX - 1), 1), :]
            anew = jax.lax.dot_general(
                alpha.astype(bf16), wbd_ref[:, :], dot_dims,
                preferred_element_type=f32) * e_i
            alpha = jnp.where(i < len_ref[:, :], anew, alpha)
        s = jax.lax.dot_general(alpha.astype(bf16), ones_ref[:, :], dot_dims,
                                preferred_element_type=f32)
        return alpha / s, acc + jnp.log(s)

    alpha0 = sexp_ref[:, :] * escr[0:1, :]
    alpha, acc = jax.lax.fori_loop(
        0, LMAX // RESCALE, step_block,
        (alpha0, jnp.zeros((1, BN), f32)))

    v = (alpha * eexp_ref[:, :]).astype(bf16)
    s_end = jax.lax.dot_general(v, ones_ref[:, :], dot_dims,
                                preferred_element_type=f32)
    zvec = jnp.log(s_end) + acc
    z16 = jax.lax.dot_general(zvec, sel_ref[:, :], dot_dims,
                              preferred_element_type=f32)
    out_ref[:, :] = scores - z16


@jax.jit
def kernel(emissions, tags, cu_seqlens, transitions, start_transitions,
           end_transitions):
    f32 = jnp.float32
    em = emissions[:, 0, :].astype(f32)
    em_pad = jnp.pad(em, ((0, LMAX), (0, 0)))
    tags_t = tags[:, 0:1].astype(jnp.int32)
    src_t = jnp.concatenate([tags_t[:1], tags_t[:-1]], axis=0)
    t2 = transitions[0].astype(f32)
    start_row = start_transitions.astype(f32)
    end_row = end_transitions.astype(f32)

    eye = jnp.eye(B, dtype=f32)
    wbd = jnp.kron(eye, jnp.exp(t2)).astype(jnp.bfloat16)
    onesbd = jnp.kron(eye, jnp.ones((N, N), f32)).astype(jnp.bfloat16)
    sel = (jax.lax.broadcasted_iota(jnp.int32, (BN, B), 0)
           == N * jax.lax.broadcasted_iota(jnp.int32, (BN, B), 1)).astype(f32)
    lengths = cu_seqlens[1:] - cu_seqlens[:-1]
    len_vec = jnp.repeat(lengths, N)[None, :].astype(jnp.int32)
    sexp = jnp.tile(jnp.exp(start_row[0]), B)[None, :]
    eexp = jnp.tile(jnp.exp(end_row[0]), B)[None, :]

    full = lambda shape: pl.BlockSpec(shape, lambda i, cu: (0,) * len(shape))
    out = pl.pallas_call(
        _crf_body,
        grid_spec=pltpu.PrefetchScalarGridSpec(
            num_scalar_prefetch=1,
            grid=(1,),
            in_specs=[
                full((T + LMAX, N)),
                full((T, 1)),
                full((T, 1)),
                full((N, N)),
                full((1, N)),
                full((1, N)),
                full((BN, BN)),
                full((BN, BN)),
                full((BN, B)),
                full((1, BN)),
                full((1, BN)),
                full((1, BN)),
            ],
            out_specs=full((1, B)),
            scratch_shapes=[pltpu.VMEM((LMAX, BN), f32)],
        ),
        out_shape=jax.ShapeDtypeStruct((1, B), f32),
    )(cu_seqlens.astype(jnp.int32), em_pad, tags_t, src_t, t2, start_row,
      end_row, wbd, onesbd, sel, len_vec, sexp, eexp)
    return out.reshape(B, 1)


# speedup vs baseline: 18.2030x; 2.8158x over previous
import jax
import jax.numpy as jnp
from jax.experimental import pallas as pl
from jax.experimental.pallas import tpu as pltpu

B = 16
T = 16384
N = 32
LMAX = 2048
BN = B * N
G = 8
CHL = LMAX // G
RO = G * N
RQ = LMAX // 4
RESCALE = 8


def _crf_body(cu_ref, em_ref, em128_ref, tags128_ref, src128_ref, w4bd_ref,
              start128_ref, end128_ref, wbd_ref, ones_ref, ones256_ref,
              sel_ref, len_ref, sexp_ref, eexp_ref, out_ref, escr):
    f32 = jnp.float32
    bf16 = jnp.bfloat16
    dims = (((1,), (0,)), ((), ()))

    lane32 = jax.lax.broadcasted_iota(jnp.int32, (RQ, 128), 1) % N
    posrel = (jax.lax.broadcasted_iota(jnp.int32, (RQ, 1), 0) * 4
              + jax.lax.broadcasted_iota(jnp.int32, (RQ, 128), 1) // N)
    isf = posrel == 0
    b_lane = jax.lax.broadcasted_iota(jnp.int32, (1, B), 1)

    scores = jnp.zeros((1, B), f32)
    for b in range(B):
        cu_b = cu_ref[b]
        len_b = cu_ref[b + 1] - cu_b
        ev = jnp.exp(em_ref[pl.ds(cu_b, LMAX), :]).astype(bf16)
        for g in range(G):
            escr[0:CHL, g, b * N:(b + 1) * N] = ev[g * CHL:(g + 1) * CHL]
        emc = em128_ref[pl.ds(cu_b // 4, RQ), :]
        tg = tags128_ref[pl.ds(cu_b // 4, RQ), :]
        sr = src128_ref[pl.ds(cu_b // 4, RQ), :]
        oh_tag = tg == lane32
        oh_src = (sr == lane32).astype(bf16)
        trans_row = jax.lax.dot_general(
            oh_src, w4bd_ref[:, :], dims, preferred_element_type=f32)
        val = (emc + jnp.where(isf, start128_ref[:, :], trans_row)
               + jnp.where(posrel == len_b - 1, end128_ref[:, :], 0.0))
        val = jnp.where(oh_tag & (posrel < len_b), val, 0.0)
        tot = jnp.sum(val, axis=(0, 1), keepdims=True)
        scores = scores + jnp.where(b_lane == b,
                                    jnp.broadcast_to(tot, (1, B)), 0.0)

    escr[CHL:CHL + 1, 0:G - 1, :] = escr[0:1, 1:G, :]
    escr[CHL:CHL + 1, G - 1:G, :] = escr[CHL - 1:CHL, G - 1:G, :]

    gi_row = jax.lax.broadcasted_iota(jnp.int32, (RO, 1), 0) // N
    thresh = len_ref[:, :] - (gi_row * CHL + 1)
    ii = jax.lax.broadcasted_iota(jnp.int32, (RO, BN), 0) % N
    jj = jax.lax.broadcasted_iota(jnp.int32, (RO, BN), 1) % N
    x0 = (ii == jj).astype(bf16)

    def step_block(o, carry):
        x, acc = carry
        for u in range(RESCALE):
            t = o * RESCALE + u
            eg = escr[pl.ds(t + 1, 1), :, :]
            eb = jnp.broadcast_to(eg.reshape(G, 1, BN),
                                  (G, N, BN)).reshape(RO, BN)
            xn = jax.lax.dot_general(
                x, wbd_ref[:, :], dims,
                preferred_element_type=f32).astype(bf16) * eb
            x = jnp.where(t < thresh, xn, x)
        rs = jax.lax.dot_general(x, ones_ref[:, :], dims,
                                 preferred_element_type=f32)
        s = jax.lax.dot_general(ones256_ref[:, :], rs.astype(bf16), dims,
                                preferred_element_type=f32)
        return (x.astype(f32) / s).astype(bf16), acc + jnp.log(s)

    x, acc = jax.lax.fori_loop(
        0, CHL // RESCALE, step_block, (x0, jnp.zeros((RO, BN), f32)))

    rowmask0 = (jax.lax.broadcasted_iota(jnp.int32, (RO, 1), 0) % N) == 0
    accsum = jnp.sum(jnp.where(rowmask0, acc, 0.0), axis=0, keepdims=True)
    f = x[0:N, :]
    acc_f = jnp.zeros((1, BN), f32)
    for g in range(1, G):
        pg = x[g * N:(g + 1) * N, :]
        pg_exp = (jnp.broadcast_to(pg[None], (B, N, BN)).reshape(BN, BN)
                  * ones_ref[:, :])
        f = jax.lax.dot_general(
            f, pg_exp, dims, preferred_element_type=f32).astype(bf16)
        rs_f = jax.lax.dot_general(f, ones_ref[:, :], dims,
                                   preferred_element_type=f32)
        s_f = jnp.sum(rs_f, axis=0, keepdims=True)
        f = (f.astype(f32) / s_f).astype(bf16)
        acc_f = acc_f + jnp.log(s_f)

    v = (f.astype(f32) * eexp_ref[:, :]).astype(bf16)
    v_exp = (jnp.broadcast_to(v[None], (B, N, BN)).reshape(BN, BN)
             * ones_ref[:, :])
    alpha0 = (sexp_ref[:, :] * escr[0:1, 0, :].astype(f32)).astype(bf16)
    y = jax.lax.dot_general(alpha0, v_exp, dims, preferred_element_type=f32)
    s_end = jax.lax.dot_general(y.astype(bf16), ones_ref[:, :], dims,
                                preferred_element_type=f32)
    zvec = jnp.log(s_end) + accsum + acc_f
    z16 = jax.lax.dot_general(zvec, sel_ref[:, :], dims,
                              preferred_element_type=f32)
    out_ref[:, :] = scores - z16


@jax.jit
def kernel(emissions, tags, cu_seqlens, transitions, start_transitions,
           end_transitions):
    f32 = jnp.float32
    bf16 = jnp.bfloat16
    em = emissions[:, 0, :].astype(f32)
    em_pad = jnp.pad(em, ((0, LMAX), (0, 0)))
    em128 = em_pad.reshape((T + LMAX) // 4, 128)
    tags_t = tags[:, 0:1].astype(jnp.int32)
    src_t = jnp.concatenate([tags_t[:1], tags_t[:-1]], axis=0)
    tags_pad = jnp.pad(tags_t, ((0, LMAX), (0, 0)))
    src_pad = jnp.pad(src_t, ((0, LMAX), (0, 0)))
    tags128 = jnp.repeat(tags_pad.reshape((T + LMAX) // 4, 4), N, axis=1)
    src128 = jnp.repeat(src_pad.reshape((T + LMAX) // 4, 4), N, axis=1)
    t2 = transitions[0].astype(f32)
    start_row = start_transitions.astype(f32)
    end_row = end_transitions.astype(f32)

    w4bd = jnp.kron(jnp.eye(4, dtype=f32), t2).astype(bf16)
    start128 = jnp.tile(start_row[0], 4)[None, :]
    end128 = jnp.tile(end_row[0], 4)[None, :]
    eyeb = jnp.eye(B, dtype=f32)
    wbd = jnp.kron(eyeb, jnp.exp(t2)).astype(bf16)
    onesbd = jnp.kron(eyeb, jnp.ones((N, N), f32)).astype(bf16)
    ones256 = jnp.kron(jnp.eye(G, dtype=f32),
                       jnp.ones((N, N), f32)).astype(bf16)
    sel = (jax.lax.broadcasted_iota(jnp.int32, (BN, B), 0)
           == N * jax.lax.broadcasted_iota(jnp.int32, (BN, B), 1)).astype(f32)
    lengths = cu_seqlens[1:] - cu_seqlens[:-1]
    len_vec = jnp.repeat(lengths, N)[None, :].astype(jnp.int32)
    sexp = jnp.tile(jnp.exp(start_row[0]), B)[None, :]
    eexp = jnp.tile(jnp.exp(end_row[0]), B)[None, :]

    full = lambda shape: pl.BlockSpec(shape, lambda i, cu: (0,) * len(shape))
    out = pl.pallas_call(
        _crf_body,
        grid_spec=pltpu.PrefetchScalarGridSpec(
            num_scalar_prefetch=1,
            grid=(1,),
            in_specs=[
                full((T + LMAX, N)),
                full(((T + LMAX) // 4, 128)),
                full(((T + LMAX) // 4, 128)),
                full(((T + LMAX) // 4, 128)),
                full((128, 128)),
                full((1, 128)),
                full((1, 128)),
                full((BN, BN)),
                full((BN, BN)),
                full((RO, RO)),
                full((BN, B)),
                full((1, BN)),
                full((1, BN)),
                full((1, BN)),
            ],
            out_specs=full((1, B)),
            scratch_shapes=[pltpu.VMEM((CHL + 1, G, BN), bf16)],
        ),
        out_shape=jax.ShapeDtypeStruct((1, B), f32),
    )(cu_seqlens.astype(jnp.int32), em_pad, em128, tags128, src128, w4bd,
      start128, end128, wbd, onesbd, ones256, sel, len_vec, sexp, eexp)
    return out.reshape(B, 1)


# speedup vs baseline: 22.8751x; 1.2567x over previous
import jax
import jax.numpy as jnp
from jax.experimental import pallas as pl
from jax.experimental.pallas import tpu as pltpu

B = 16
T = 16384
N = 32
LMAX = 2048
BN = B * N
G = 8
CHL = LMAX // G
RO = G * N
RQ = LMAX // 4
RESCALE = 8


def _crf_body(cu_ref, em_ref, em128_ref, tags128_ref, src128_ref, w4bd_ref,
              start128_ref, end128_ref, wbd_ref, ones_ref, ones256_ref,
              sel_ref, len_ref, sexp_ref, eexp_ref, out_ref, escr):
    f32 = jnp.float32
    bf16 = jnp.bfloat16
    dims = (((1,), (0,)), ((), ()))

    lane32 = (jax.lax.broadcasted_iota(jnp.int32, (RQ, 128), 1) % N
              ).astype(bf16)
    posrel = (jax.lax.broadcasted_iota(jnp.int32, (RQ, 1), 0) * 4
              + jax.lax.broadcasted_iota(jnp.int32, (RQ, 128), 1) // N
              ).astype(f32)
    isf = posrel == 0.0
    b_lane = jax.lax.broadcasted_iota(jnp.int32, (1, B), 1)

    scores = jnp.zeros((1, B), f32)
    for b in range(B):
        cu_b = cu_ref[b]
        len_b = (cu_ref[b + 1] - cu_b).astype(f32)
        ev = jnp.exp(em_ref[pl.ds(pl.multiple_of(cu_b, 8), LMAX), :]
                     ).astype(bf16)
        escr[:, :, b * N:(b + 1) * N] = ev.reshape(G, CHL, N)
        r0 = pl.multiple_of(cu_b // 4, 8)
        emc = em128_ref[pl.ds(r0, RQ), :]
        tg = tags128_ref[pl.ds(r0, RQ), :]
        sr = src128_ref[pl.ds(r0, RQ), :]
        oh_tag = tg == lane32
        oh_src = (sr == lane32).astype(bf16)
        trans_row = jax.lax.dot_general(
            oh_src, w4bd_ref[:, :], dims, preferred_element_type=f32)
        val = (emc + jnp.where(isf, start128_ref[:, :], trans_row)
               + jnp.where(posrel == len_b - 1.0, end128_ref[:, :], 0.0))
        val = jnp.where(oh_tag & (posrel < len_b), val, 0.0)
        tot = jnp.sum(val, axis=(0, 1), keepdims=True)
        scores = scores + jnp.where(b_lane == b,
                                    jnp.broadcast_to(tot, (1, B)), 0.0)

    gi_row = jax.lax.broadcasted_iota(jnp.int32, (RO, 1), 0) // N
    thresh_i = len_ref[:, :] - gi_row * CHL
    thresh = thresh_i.astype(bf16)
    m0 = (gi_row > 0) & (thresh_i > 0)
    ii = jax.lax.broadcasted_iota(jnp.int32, (RO, BN), 0) % N
    jj = jax.lax.broadcasted_iota(jnp.int32, (RO, BN), 1) % N
    x0 = (ii == jj).astype(bf16)

    def bcast(row8):
        return jnp.broadcast_to(row8.reshape(G, 1, BN), (G, N, BN)
                                ).reshape(RO, BN)

    def advance(x, eb):
        return jax.lax.dot_general(
            x, wbd_ref[:, :], dims,
            preferred_element_type=jnp.float32).astype(bf16) * eb

    def rescale(x, acc):
        rs = jax.lax.dot_general(x, ones_ref[:, :], dims,
                                 preferred_element_type=f32)
        s = jax.lax.dot_general(ones256_ref[:, :], rs.astype(bf16), dims,
                                preferred_element_type=f32)
        return (x.astype(f32) / s).astype(bf16), acc + jnp.log(s)

    x = x0
    slab0 = escr[:, 0:RESCALE, :]
    for u in range(RESCALE):
        xn = advance(x, bcast(slab0[:, u, :]))
        if u == 0:
            x = jnp.where(m0, xn, x)
        else:
            x = jnp.where(float(u) < thresh, xn, x)
    x, acc = rescale(x, jnp.zeros((RO, BN), f32))

    def step_block(o, carry):
        x, acc = carry
        slab = escr[:, pl.ds(pl.multiple_of(o * RESCALE, 8), RESCALE), :]
        for u in range(RESCALE):
            t = (o * RESCALE + u).astype(bf16)
            xn = advance(x, bcast(slab[:, u, :]))
            x = jnp.where(t < thresh, xn, x)
        return rescale(x, acc)

    x, acc = jax.lax.fori_loop(1, CHL // RESCALE, step_block, (x, acc))

    rowmask0 = (jax.lax.broadcasted_iota(jnp.int32, (RO, 1), 0) % N) == 0
    accsum = jnp.sum(jnp.where(rowmask0, acc, 0.0), axis=0, keepdims=True)
    f = x[0:N, :]
    acc_f = jnp.zeros((1, BN), f32)
    for g in range(1, G):
        pg = x[g * N:(g + 1) * N, :]
        pg_exp = (jnp.broadcast_to(pg[None], (B, N, BN)).reshape(BN, BN)
                  * ones_ref[:, :])
        f = jax.lax.dot_general(
            f, pg_exp, dims, preferred_element_type=f32).astype(bf16)
        rs_f = jax.lax.dot_general(f, ones_ref[:, :], dims,
                                   preferred_element_type=f32)
        s_f = jnp.sum(rs_f, axis=0, keepdims=True)
        f = (f.astype(f32) / s_f).astype(bf16)
        acc_f = acc_f + jnp.log(s_f)

    v = (f.astype(f32) * eexp_ref[:, :]).astype(bf16)
    v_exp = (jnp.broadcast_to(v[None], (B, N, BN)).reshape(BN, BN)
             * ones_ref[:, :])
    alpha0 = (sexp_ref[:, :] * escr[0, 0:1, :].astype(f32)).astype(bf16)
    y = jax.lax.dot_general(alpha0, v_exp, dims, preferred_element_type=f32)
    s_end = jax.lax.dot_general(y.astype(bf16), ones_ref[:, :], dims,
                                preferred_element_type=f32)
    zvec = jnp.log(s_end) + accsum + acc_f
    z16 = jax.lax.dot_general(zvec, sel_ref[:, :], dims,
                              preferred_element_type=f32)
    out_ref[:, :] = scores - z16


@jax.jit
def kernel(emissions, tags, cu_seqlens, transitions, start_transitions,
           end_transitions):
    f32 = jnp.float32
    bf16 = jnp.bfloat16
    em = emissions[:, 0, :].astype(f32)
    em_pad = jnp.pad(em, ((0, LMAX), (0, 0)))
    em128 = em_pad.reshape((T + LMAX) // 4, 128)
    tags_t = tags[:, 0:1].astype(jnp.int32)
    src_t = jnp.concatenate([tags_t[:1], tags_t[:-1]], axis=0)
    tags_pad = jnp.pad(tags_t, ((0, LMAX), (0, 0)))
    src_pad = jnp.pad(src_t, ((0, LMAX), (0, 0)))
    tags128 = jnp.repeat(tags_pad.reshape((T + LMAX) // 4, 4), N,
                         axis=1).astype(bf16)
    src128 = jnp.repeat(src_pad.reshape((T + LMAX) // 4, 4), N,
                        axis=1).astype(bf16)
    t2 = transitions[0].astype(f32)
    start_row = start_transitions.astype(f32)
    end_row = end_transitions.astype(f32)

    w4bd = jnp.kron(jnp.eye(4, dtype=f32), t2).astype(bf16)
    start128 = jnp.tile(start_row[0], 4)[None, :]
    end128 = jnp.tile(end_row[0], 4)[None, :]
    eyeb = jnp.eye(B, dtype=f32)
    wbd = jnp.kron(eyeb, jnp.exp(t2)).astype(bf16)
    onesbd = jnp.kron(eyeb, jnp.ones((N, N), f32)).astype(bf16)
    ones256 = jnp.kron(jnp.eye(G, dtype=f32),
                       jnp.ones((N, N), f32)).astype(bf16)
    sel = (jax.lax.broadcasted_iota(jnp.int32, (BN, B), 0)
           == N * jax.lax.broadcasted_iota(jnp.int32, (BN, B), 1)).astype(f32)
    lengths = cu_seqlens[1:] - cu_seqlens[:-1]
    len_vec = jnp.repeat(lengths, N)[None, :].astype(jnp.int32)
    sexp = jnp.tile(jnp.exp(start_row[0]), B)[None, :]
    eexp = jnp.tile(jnp.exp(end_row[0]), B)[None, :]

    full = lambda shape: pl.BlockSpec(shape, lambda i, cu: (0,) * len(shape))
    out = pl.pallas_call(
        _crf_body,
        grid_spec=pltpu.PrefetchScalarGridSpec(
            num_scalar_prefetch=1,
            grid=(1,),
            in_specs=[
                full((T + LMAX, N)),
                full(((T + LMAX) // 4, 128)),
                full(((T + LMAX) // 4, 128)),
                full(((T + LMAX) // 4, 128)),
                full((128, 128)),
                full((1, 128)),
                full((1, 128)),
                full((BN, BN)),
                full((BN, BN)),
                full((RO, RO)),
                full((BN, B)),
                full((1, BN)),
                full((1, BN)),
                full((1, BN)),
            ],
            out_specs=full((1, B)),
            scratch_shapes=[pltpu.VMEM((G, CHL, BN), bf16)],
        ),
        out_shape=jax.ShapeDtypeStruct((1, B), f32),
    )(cu_seqlens.astype(jnp.int32), em_pad, em128, tags128, src128, w4bd,
      start128, end128, wbd, onesbd, ones256, sel, len_vec, sexp, eexp)
    return out.reshape(B, 1)


# speedup vs baseline: 23.3600x; 1.0212x over previous
import jax
import jax.numpy as jnp
from jax.experimental import pallas as pl
from jax.experimental.pallas import tpu as pltpu

B = 16
T = 16384
N = 32
LMAX = 2048
BN = B * N
G = 8
CHL = LMAX // G
RO = G * N
RQ = LMAX // 4
RESCALE = 8


def _crf_body(cu_ref, em_ref, em128_ref, tags128_ref, src128_ref, w4bd_ref,
              start128_ref, end128_ref, wbd_ref, ones_ref, ones256_ref,
              sel_ref, len_ref, sexp_ref, eexp_ref, out_ref, escr):
    f32 = jnp.float32
    bf16 = jnp.bfloat16
    dims = (((1,), (0,)), ((), ()))

    lane32 = (jax.lax.broadcasted_iota(jnp.int32, (RQ, 128), 1) % N
              ).astype(bf16)
    posrel = (jax.lax.broadcasted_iota(jnp.int32, (RQ, 1), 0) * 4
              + jax.lax.broadcasted_iota(jnp.int32, (RQ, 128), 1) // N
              ).astype(f32)
    isf = posrel == 0.0
    b_lane = jax.lax.broadcasted_iota(jnp.int32, (1, B), 1)

    scores = jnp.zeros((1, B), f32)
    for b in range(B):
        cu_b = cu_ref[b]
        len_b = (cu_ref[b + 1] - cu_b).astype(f32)
        ev = jnp.exp(em_ref[pl.ds(pl.multiple_of(cu_b, 8), LMAX), :]
                     ).astype(bf16)
        escr[:, :, b * N:(b + 1) * N] = ev.reshape(G, CHL, N)
        r0 = pl.multiple_of(cu_b // 4, 8)
        emc = em128_ref[pl.ds(r0, RQ), :]
        tg = tags128_ref[pl.ds(r0, RQ), :]
        sr = src128_ref[pl.ds(r0, RQ), :]
        oh_tag = tg == lane32
        oh_src = (sr == lane32).astype(bf16)
        trans_row = jax.lax.dot_general(
            oh_src, w4bd_ref[:, :], dims, preferred_element_type=f32)
        val = (emc + jnp.where(isf, start128_ref[:, :], trans_row)
               + jnp.where(posrel == len_b - 1.0, end128_ref[:, :], 0.0))
        val = jnp.where(oh_tag & (posrel < len_b), val, 0.0)
        tot = jnp.sum(val, axis=(0, 1), keepdims=True)
        scores = scores + jnp.where(b_lane == b,
                                    jnp.broadcast_to(tot, (1, B)), 0.0)

    gi_row = jax.lax.broadcasted_iota(jnp.int32, (RO, 1), 0) // N
    thresh_i = len_ref[:, :] - gi_row * CHL
    thresh = thresh_i.astype(bf16)
    m0 = (gi_row > 0) & (thresh_i > 0)
    ii = jax.lax.broadcasted_iota(jnp.int32, (RO, BN), 0) % N
    jj = jax.lax.broadcasted_iota(jnp.int32, (RO, BN), 1) % N
    x0 = (ii == jj).astype(bf16)

    def bcast(row8):
        return jnp.broadcast_to(row8.reshape(G, 1, BN), (G, N, BN)
                                ).reshape(RO, BN)

    def advance(x, eb):
        return jax.lax.dot_general(
            x, wbd_ref[:, :], dims,
            preferred_element_type=jnp.float32).astype(bf16) * eb

    def rescale(x, acc):
        rs = jax.lax.dot_general(x, ones_ref[:, :], dims,
                                 preferred_element_type=f32)
        s = jax.lax.dot_general(ones256_ref[:, :], rs.astype(bf16), dims,
                                preferred_element_type=f32)
        return (x.astype(f32) / s).astype(bf16), acc + jnp.log(s)

    SLAB = 2 * RESCALE
    x = x0
    acc = jnp.zeros((RO, BN), f32)
    slab0 = escr[:, 0:SLAB, :]
    for u in range(SLAB):
        xn = advance(x, bcast(slab0[:, u, :]))
        if u == 0:
            x = jnp.where(m0, xn, x)
        else:
            x = jnp.where(float(u) < thresh, xn, x)
        if u % RESCALE == RESCALE - 1:
            x, acc = rescale(x, acc)

    def step_block(o, carry):
        x, acc = carry
        slab = escr[:, pl.ds(pl.multiple_of(o * SLAB, SLAB), SLAB), :]
        for u in range(SLAB):
            t = (o * SLAB + u).astype(bf16)
            xn = advance(x, bcast(slab[:, u, :]))
            x = jnp.where(t < thresh, xn, x)
            if u % RESCALE == RESCALE - 1:
                x, acc = rescale(x, acc)
        return x, acc

    x, acc = jax.lax.fori_loop(1, CHL // SLAB, step_block, (x, acc))

    rowmask0 = (jax.lax.broadcasted_iota(jnp.int32, (RO, 1), 0) % N) == 0
    accsum = jnp.sum(jnp.where(rowmask0, acc, 0.0), axis=0, keepdims=True)
    f = x[0:N, :]
    acc_f = jnp.zeros((1, BN), f32)
    for g in range(1, G):
        pg = x[g * N:(g + 1) * N, :]
        pg_exp = (jnp.broadcast_to(pg[None], (B, N, BN)).reshape(BN, BN)
                  * ones_ref[:, :])
        f = jax.lax.dot_general(
            f, pg_exp, dims, preferred_element_type=f32).astype(bf16)
        rs_f = jax.lax.dot_general(f, ones_ref[:, :], dims,
                                   preferred_element_type=f32)
        s_f = jnp.sum(rs_f, axis=0, keepdims=True)
        f = (f.astype(f32) / s_f).astype(bf16)
        acc_f = acc_f + jnp.log(s_f)

    v = (f.astype(f32) * eexp_ref[:, :]).astype(bf16)
    v_exp = (jnp.broadcast_to(v[None], (B, N, BN)).reshape(BN, BN)
             * ones_ref[:, :])
    alpha0 = (sexp_ref[:, :] * escr[0, 0:1, :].astype(f32)).astype(bf16)
    y = jax.lax.dot_general(alpha0, v_exp, dims, preferred_element_type=f32)
    s_end = jax.lax.dot_general(y.astype(bf16), ones_ref[:, :], dims,
                                preferred_element_type=f32)
    zvec = jnp.log(s_end) + accsum + acc_f
    z16 = jax.lax.dot_general(zvec, sel_ref[:, :], dims,
                              preferred_element_type=f32)
    out_ref[:, :] = scores - z16


@jax.jit
def kernel(emissions, tags, cu_seqlens, transitions, start_transitions,
           end_transitions):
    f32 = jnp.float32
    bf16 = jnp.bfloat16
    em = emissions[:, 0, :].astype(f32)
    em_pad = jnp.pad(em, ((0, LMAX), (0, 0)))
    em128 = em_pad.reshape((T + LMAX) // 4, 128)
    tags_t = tags[:, 0:1].astype(jnp.int32)
    src_t = jnp.concatenate([tags_t[:1], tags_t[:-1]], axis=0)
    tags_pad = jnp.pad(tags_t, ((0, LMAX), (0, 0)))
    src_pad = jnp.pad(src_t, ((0, LMAX), (0, 0)))
    tags128 = jnp.repeat(tags_pad.reshape((T + LMAX) // 4, 4), N,
                         axis=1).astype(bf16)
    src128 = jnp.repeat(src_pad.reshape((T + LMAX) // 4, 4), N,
                        axis=1).astype(bf16)
    t2 = transitions[0].astype(f32)
    start_row = start_transitions.astype(f32)
    end_row = end_transitions.astype(f32)

    w4bd = jnp.kron(jnp.eye(4, dtype=f32), t2).astype(bf16)
    start128 = jnp.tile(start_row[0], 4)[None, :]
    end128 = jnp.tile(end_row[0], 4)[None, :]
    eyeb = jnp.eye(B, dtype=f32)
    wbd = jnp.kron(eyeb, jnp.exp(t2)).astype(bf16)
    onesbd = jnp.kron(eyeb, jnp.ones((N, N), f32)).astype(bf16)
    ones256 = jnp.kron(jnp.eye(G, dtype=f32),
                       jnp.ones((N, N), f32)).astype(bf16)
    sel = (jax.lax.broadcasted_iota(jnp.int32, (BN, B), 0)
           == N * jax.lax.broadcasted_iota(jnp.int32, (BN, B), 1)).astype(f32)
    lengths = cu_seqlens[1:] - cu_seqlens[:-1]
    len_vec = jnp.repeat(lengths, N)[None, :].astype(jnp.int32)
    sexp = jnp.tile(jnp.exp(start_row[0]), B)[None, :]
    eexp = jnp.tile(jnp.exp(end_row[0]), B)[None, :]

    full = lambda shape: pl.BlockSpec(shape, lambda i, cu: (0,) * len(shape))
    out = pl.pallas_call(
        _crf_body,
        grid_spec=pltpu.PrefetchScalarGridSpec(
            num_scalar_prefetch=1,
            grid=(1,),
            in_specs=[
                full((T + LMAX, N)),
                full(((T + LMAX) // 4, 128)),
                full(((T + LMAX) // 4, 128)),
                full(((T + LMAX) // 4, 128)),
                full((128, 128)),
                full((1, 128)),
                full((1, 128)),
                full((BN, BN)),
                full((BN, BN)),
                full((RO, RO)),
                full((BN, B)),
                full((1, BN)),
                full((1, BN)),
                full((1, BN)),
            ],
            out_specs=full((1, B)),
            scratch_shapes=[pltpu.VMEM((G, CHL, BN), bf16)],
        ),
        out_shape=jax.ShapeDtypeStruct((1, B), f32),
    )(cu_seqlens.astype(jnp.int32), em_pad, em128, tags128, src128, w4bd,
      start128, end128, wbd, onesbd, ones256, sel, len_vec, sexp, eexp)
    return out.reshape(B, 1)
